# baseline (speedup 1.0000x reference)
import functools
import math

import jax
import jax.numpy as jnp
from jax import lax
from jax.experimental import pallas as pl
from jax.experimental.pallas import tpu as pltpu
from jax.experimental.pallas import tpu_sc as plsc

NUM_SPECIES = 100
EMBED_DIM = 8
NUM_BASIS = 16
CUTOFF = 5.0
N_NODES = 100000
N_EDGES = 1600000


_NC, _NS = 2, 16
_NW = _NC * _NS
_PER_W = 3200
_N_PAD = _NW * _PER_W
_WIDTH = 2 * EMBED_DIM


def _node_gather_body(x_hbm, w_hbm, outx_hbm, outz_hbm, idx_v, tab_v, rxt_v, rzt_v, sem):
    wid = lax.axis_index("s") * _NC + lax.axis_index("c")
    base = wid * _PER_W
    h_idx = pltpu.async_copy(x_hbm.at[pl.ds(base, _PER_W)], idx_v, sem)
    pltpu.sync_copy(w_hbm, tab_v)
    h_idx.wait()

    def group(g, _):
        idx16 = idx_v[pl.ds(g * 16, 16)]
        fbase = idx16 * _WIDTH
        for j in range(_WIDTH):
            vals = plsc.load_gather(tab_v, [fbase + j])
            buf = rxt_v if j < EMBED_DIM else rzt_v
            buf[j % EMBED_DIM, pl.ds(g * 16, 16)] = vals
        return 0

    lax.fori_loop(0, _PER_W // 16, group, 0)
    handles = []
    for j in range(EMBED_DIM):
        handles.append(pltpu.async_copy(rxt_v.at[j], outx_hbm.at[j, pl.ds(base, _PER_W)], sem))
        handles.append(pltpu.async_copy(rzt_v.at[j], outz_hbm.at[j, pl.ds(base, _PER_W)], sem))
    for h in handles:
        h.wait()


@functools.cache
def _node_gather():
    return pl.kernel(
        _node_gather_body,
        mesh=plsc.VectorSubcoreMesh(core_axis_name="c", subcore_axis_name="s"),
        compiler_params=pltpu.CompilerParams(needs_layout_passes=False),
        out_type=[
            jax.ShapeDtypeStruct((EMBED_DIM, _N_PAD), jnp.float32),
            jax.ShapeDtypeStruct((EMBED_DIM, _N_PAD), jnp.float32),
        ],
        scratch_types=[
            pltpu.VMEM((_PER_W,), jnp.int32),
            pltpu.VMEM((NUM_SPECIES * _WIDTH,), jnp.float32),
            pltpu.VMEM((EMBED_DIM, _PER_W), jnp.float32),
            pltpu.VMEM((EMBED_DIM, _PER_W), jnp.float32),
            pltpu.SemaphoreType.DMA,
        ],
    )


_NR = 8
_E_PAD = 1638400
_RANGE = _E_PAD // _NR
_EDGE_BLK = 6400


def _edge_body(e_ref, o_ref):
    e = e_ref[...]
    xr = e[0:_NR, :]
    yr = e[_NR:2 * _NR, :]
    zr = e[2 * _NR:3 * _NR, :]
    r2 = xr * xr + yr * yr + zr * zr
    r = jnp.sqrt(r2)
    theta = r * (math.pi / CUTOFF)
    q = jnp.round(theta * (2.0 / math.pi))
    t = theta - q * (math.pi / 2.0)
    t2 = t * t
    st = t * (1.0 + t2 * (-1.0 / 6.0 + t2 * (1.0 / 120.0 + t2 * (-1.0 / 5040.0))))
    ct = 1.0 + t2 * (-0.5 + t2 * (1.0 / 24.0 + t2 * (-1.0 / 720.0 + t2 * (1.0 / 40320.0))))
    qm = jnp.bitwise_and(q.astype(jnp.int32), 3)
    bit0 = jnp.bitwise_and(qm, 1) == 1
    sin_sign = jnp.where(qm >= 2, -1.0, 1.0)
    cos_sign = jnp.where(jnp.logical_or(qm == 1, qm == 2), -1.0, 1.0)
    sin1 = sin_sign * jnp.where(bit0, ct, st)
    cos1 = cos_sign * jnp.where(bit0, st, ct)
    s1 = (math.sqrt(2.0 / CUTOFF) / r) * sin1
    c2x = 2.0 * cos1
    s_pp = jnp.zeros_like(s1)
    s_p = s1
    o_ref[pl.ds(0, _NR), :] = s1
    for n in range(1, NUM_BASIS):
        s_n = c2x * s_p - s_pp
        o_ref[pl.ds(n * _NR, _NR), :] = s_n
        s_pp, s_p = s_p, s_n


def _edge_call(e24):
    grid = _RANGE // _EDGE_BLK
    return pl.pallas_call(
        _edge_body,
        grid=(grid,),
        in_specs=[pl.BlockSpec((3 * _NR, _EDGE_BLK), lambda i: (0, i))],
        out_specs=pl.BlockSpec((NUM_BASIS * _NR, _EDGE_BLK), lambda i: (0, i)),
        out_shape=jax.ShapeDtypeStruct((NUM_BASIS * _NR, _RANGE), jnp.float32),
    )(e24)


def kernel(x, edge_attr, W_x, W_z):
    w_flat = jnp.concatenate([W_x, W_z], axis=1).reshape(-1)
    x_pad = jnp.pad(x.astype(jnp.int32), (0, _N_PAD - N_NODES))
    fxt, fzt = _node_gather()(x_pad, w_flat)
    h_node_x = jnp.transpose(fxt)[:N_NODES]
    h_node_z = jnp.transpose(fzt)[:N_NODES]
    e_t = jnp.pad(jnp.transpose(edge_attr), ((0, 0), (0, _E_PAD - N_EDGES)))
    h128 = _edge_call(e_t.reshape(3 * _NR, _RANGE))
    h_edge = jnp.transpose(h128.reshape(NUM_BASIS, _E_PAD)[:, :N_EDGES])
    return (h_node_x, h_node_z, h_edge)


# speedup vs baseline: 19.5446x; 19.5446x over previous
import functools
import math

import jax
import jax.numpy as jnp
from jax import lax
from jax.experimental import pallas as pl
from jax.experimental.pallas import tpu as pltpu
from jax.experimental.pallas import tpu_sc as plsc

NUM_SPECIES = 100
EMBED_DIM = 8
NUM_BASIS = 16
CUTOFF = 5.0
N_NODES = 100000
N_EDGES = 1600000


_NC, _NS = 2, 16
_NW = _NC * _NS
_PER_W = 3200
_N_PAD = _NW * _PER_W
_WIDTH = 2 * EMBED_DIM


def _node_gather_body(x_hbm, w_hbm, outx_hbm, outz_hbm, idx_v, tab_v, rxt_v, rzt_v, sem):
    wid = lax.axis_index("s") * _NC + lax.axis_index("c")
    base = wid * _PER_W
    h_idx = pltpu.async_copy(x_hbm.at[pl.ds(base, _PER_W)], idx_v, sem)
    pltpu.sync_copy(w_hbm, tab_v)
    h_idx.wait()

    def group(g, _):
        idx16 = idx_v[pl.ds(g * 16, 16)]
        fbase = idx16 * _WIDTH
        for j in range(_WIDTH):
            vals = plsc.load_gather(tab_v, [fbase + j])
            buf = rxt_v if j < EMBED_DIM else rzt_v
            buf[j % EMBED_DIM, pl.ds(g * 16, 16)] = vals
        return 0

    lax.fori_loop(0, _PER_W // 16, group, 0)
    handles = []
    for j in range(EMBED_DIM):
        handles.append(pltpu.async_copy(rxt_v.at[j], outx_hbm.at[j, pl.ds(base, _PER_W)], sem))
        handles.append(pltpu.async_copy(rzt_v.at[j], outz_hbm.at[j, pl.ds(base, _PER_W)], sem))
    for h in handles:
        h.wait()


@functools.cache
def _node_gather():
    return pl.kernel(
        _node_gather_body,
        mesh=plsc.VectorSubcoreMesh(core_axis_name="c", subcore_axis_name="s"),
        compiler_params=pltpu.CompilerParams(needs_layout_passes=False),
        out_type=[
            jax.ShapeDtypeStruct((EMBED_DIM, _N_PAD), jnp.float32),
            jax.ShapeDtypeStruct((EMBED_DIM, _N_PAD), jnp.float32),
        ],
        scratch_types=[
            pltpu.VMEM((_PER_W,), jnp.int32),
            pltpu.VMEM((NUM_SPECIES * _WIDTH,), jnp.float32),
            pltpu.VMEM((EMBED_DIM, _PER_W), jnp.float32),
            pltpu.VMEM((EMBED_DIM, _PER_W), jnp.float32),
            pltpu.SemaphoreType.DMA,
        ],
    )


_EDGE_BLK = 12800


def _edge_body(e_ref, o_ref):
    e = e_ref[...]
    xr = e[0:1, :]
    yr = e[1:2, :]
    zr = e[2:3, :]
    r2 = xr * xr + yr * yr + zr * zr
    r = jnp.sqrt(r2)
    theta = r * (math.pi / CUTOFF)
    q = jnp.round(theta * (2.0 / math.pi))
    t = theta - q * (math.pi / 2.0)
    t2 = t * t
    st = t * (1.0 + t2 * (-1.0 / 6.0 + t2 * (1.0 / 120.0 + t2 * (-1.0 / 5040.0))))
    ct = 1.0 + t2 * (-0.5 + t2 * (1.0 / 24.0 + t2 * (-1.0 / 720.0 + t2 * (1.0 / 40320.0))))
    qm = jnp.bitwise_and(q.astype(jnp.int32), 3)
    bit0 = jnp.bitwise_and(qm, 1) == 1
    sin_sign = jnp.where(qm >= 2, -1.0, 1.0)
    cos_sign = jnp.where(jnp.logical_or(qm == 1, qm == 2), -1.0, 1.0)
    sin1 = sin_sign * jnp.where(bit0, ct, st)
    cos1 = cos_sign * jnp.where(bit0, st, ct)
    s1 = (math.sqrt(2.0 / CUTOFF) / r) * sin1
    c2x = 2.0 * cos1
    s_pp = jnp.zeros_like(s1)
    s_p = s1
    o_ref[pl.ds(0, 1), :] = s1
    for n in range(1, NUM_BASIS):
        s_n = c2x * s_p - s_pp
        o_ref[pl.ds(n, 1), :] = s_n
        s_pp, s_p = s_p, s_n


def _edge_call(edge_attr_t):
    grid = N_EDGES // _EDGE_BLK
    return pl.pallas_call(
        _edge_body,
        grid=(grid,),
        in_specs=[pl.BlockSpec((3, _EDGE_BLK), lambda i: (0, i))],
        out_specs=pl.BlockSpec((NUM_BASIS, _EDGE_BLK), lambda i: (0, i)),
        out_shape=jax.ShapeDtypeStruct((NUM_BASIS, N_EDGES), jnp.float32),
    )(edge_attr_t)


def kernel(x, edge_attr, W_x, W_z):
    w_flat = jnp.concatenate([W_x, W_z], axis=1).reshape(-1)
    x_pad = jnp.pad(x.astype(jnp.int32), (0, _N_PAD - N_NODES))
    fxt, fzt = _node_gather()(x_pad, w_flat)
    h_node_x = jnp.transpose(fxt)[:N_NODES]
    h_node_z = jnp.transpose(fzt)[:N_NODES]
    h_edge = jnp.transpose(_edge_call(jnp.transpose(edge_attr)))
    return (h_node_x, h_node_z, h_edge)
